# Initial kernel scaffold; baseline (speedup 1.0000x reference)
import jax
import jax.numpy as jnp
from jax.experimental import pallas as pl


def kernel(point_cloud, category_ids, W1, b1, g1, bb1, W2, b2, g2, bb2, W3, b3, g3, bb3, W4, b4, g4, bb4, W5, b5, g5, bb5, W6, b6, g6, bb6, W7, b7, g7, bb7, cat_bias):
    raise NotImplementedError("write your pallas kernel here")


# speedup vs baseline: 14.7566x; 14.7566x over previous
import jax
import jax.numpy as jnp
from jax import lax
from jax.experimental import pallas as pl
from jax.experimental.pallas import tpu as pltpu

F32 = jnp.float32
K_NN = 16


HI = lax.Precision.HIGHEST
BF = jnp.bfloat16


def _dgT(a, b):
    return lax.dot_general(a.astype(BF), b.astype(BF),
                           (((1,), (1,)), ((), ())),
                           preferred_element_type=F32)


def _dgT32(a, b):
    return lax.dot_general(a, b, (((1,), (1,)), ((), ())),
                           preferred_element_type=F32, precision=HI)


def _mean0(h):
    s = h
    while s.shape[0] > 1:
        w = s.shape[0] // 2
        s = lax.slice(s, (0, 0), (w, s.shape[1])) \
            + lax.slice(s, (w, 0), (2 * w, s.shape[1]))
    return s * (1.0 / h.shape[0])


def _bn_act(h, g, bb):
    mu = _mean0(h)
    var = _mean0(jnp.square(h - mu))
    return jnp.maximum((h - mu) / jnp.sqrt(var + 1e-5) * g + bb, 0.0)


def _k1_body(x_ref, w1_ref, b1_ref, g1_ref, bb1_ref,
             w2_ref, b2_ref, g2_ref, bb2_ref, lf_ref):
    h = _dgT(x_ref[...], w1_ref[...]) + b1_ref[...]
    h = _bn_act(h, g1_ref[...], bb1_ref[...])
    h = _dgT(h, w2_ref[...]) + b2_ref[...]
    lf_ref[...] = _bn_act(h, g2_ref[...], bb2_ref[...])


def _k2_body(lf_ref, ef_ref, d_ref):
    lf = lf_ref[...]
    lsq = lf * lf
    sqc = jnp.sum(lsq, axis=1, keepdims=True)
    ones = jnp.ones((1, lf.shape[1]), F32)
    sqr = _dgT32(ones, lsq)
    gram = _dgT(lf, lf)
    d_ref[...] = sqc + sqr - 2.0 * gram
    ef_ref[...] = jnp.full(ef_ref.shape, -jnp.inf, F32)

    def body(r, c):
        d = d_ref[...]
        m = jnp.min(d, axis=1, keepdims=True)
        msk = d <= m
        d_ref[...] = jnp.where(msk, jnp.inf, d)
        sel = lax.dot_general(msk.astype(BF), lf.astype(BF),
                              (((1,), (0,)), ((), ())),
                              preferred_element_type=F32)
        ef_ref[...] = jnp.maximum(ef_ref[...], sel)
        return c

    lax.fori_loop(0, K_NN, body, 0)


def _k3a_body(ids_ref, lf_ref, ef_ref,
              w3a_ref, w3b_ref, b3_ref, g3_ref, bb3_ref,
              w4_ref, b4_ref, g4_ref, bb4_ref,
              w5_ref, b5_ref, g5_ref, bb5_ref,
              w6_ref, b6_ref, g6_ref, bb6_ref,
              w7a_ref, w7b_ref, b7_ref, cb_ref,
              el_ref, v_ref, mu_ref, rstd_ref, cbm_ref):
    BN_, _ = el_ref.shape
    B = v_ref.shape[0]
    N = BN_ // B
    lf = lf_ref[...]
    ef = ef_ref[...]
    h = _dgT(lf, w3a_ref[...]) + _dgT(ef, w3b_ref[...]) + b3_ref[...]
    el = _bn_act(h, g3_ref[...], bb3_ref[...])
    el = _bn_act(_dgT(el, w4_ref[...]) + b4_ref[...], g4_ref[...], bb4_ref[...])
    el_ref[...] = el
    mx, sm = [], []
    for b in range(B):
        blk = lax.slice_in_dim(el, b * N, (b + 1) * N, axis=0)
        mx.append(jnp.max(blk, axis=0, keepdims=True))
        sm.append(jnp.sum(blk, axis=0, keepdims=True))
    gin = jnp.concatenate(mx, axis=0)
    s = jnp.concatenate(sm, axis=0)
    gf = _bn_act(_dgT(gin, w5_ref[...]) + b5_ref[...], g5_ref[...], bb5_ref[...])
    gf = _bn_act(_dgT(gf, w6_ref[...]) + b6_ref[...], g6_ref[...], bb6_ref[...])
    w7a = w7a_ref[...]
    v = _dgT(gf, w7b_ref[...]) + b7_ref[...]
    v_ref[...] = v
    M = float(BN_)
    mu = _dgT32(jnp.sum(s, axis=0, keepdims=True) / M, w7a) \
        + jnp.mean(v, axis=0, keepdims=True)
    C = lax.dot_general(el, el, (((0,), (0,)), ((), ())),
                        preferred_element_type=F32, precision=HI)
    q = jnp.sum(_dgT32(w7a, C) * w7a, axis=1)[None, :] / M
    P = _dgT32(s, w7a)
    r = jnp.sum(v * P, axis=0, keepdims=True) / M
    t = jnp.sum(v * v, axis=0, keepdims=True) * (N / M)
    var = q + 2.0 * r + t - mu * mu
    mu_ref[...] = mu
    rstd_ref[...] = 1.0 / jnp.sqrt(var + 1e-5)
    cbrows = [cb_ref[pl.ds(ids_ref[b], 1), :] for b in range(B)]
    cbm_ref[...] = jnp.concatenate(cbrows, axis=0)


def _k3b_body(el_ref, v_ref, mu_ref, rstd_ref, g7_ref, bb7_ref, w7a_ref,
              cbm_ref, out_ref):
    N, O = out_ref.shape
    h = _dgT(el_ref[...], w7a_ref[...]) \
        + jnp.broadcast_to(v_ref[...].reshape(1, O), (N, O))
    y = (h - mu_ref[...]) * rstd_ref[...] * g7_ref[...] + bb7_ref[...]
    y = jnp.maximum(y, 0.0)
    out_ref[...] = y + 0.1 * jnp.broadcast_to(cbm_ref[...].reshape(1, O),
                                              (N, O))


def _row(v):
    return v.reshape(1, -1)


def _run_k1(x, W1, b1, g1, bb1, W2, b2, g2, bb2):
    BN_ = x.shape[0]
    D = W2.shape[0]
    row = _row
    return pl.pallas_call(
        _k1_body,
        out_shape=jax.ShapeDtypeStruct((BN_, D), F32),
    )(x, W1, row(b1), row(g1), row(bb1), W2, row(b2), row(g2), row(bb2))


def _run_k2(lf, B):
    BN_, D = lf.shape
    N = BN_ // B
    return pl.pallas_call(
        _k2_body,
        grid=(B,),
        in_specs=[pl.BlockSpec((N, D), lambda b: (b, 0))],
        out_specs=pl.BlockSpec((N, D), lambda b: (b, 0)),
        out_shape=jax.ShapeDtypeStruct((BN_, D), F32),
        scratch_shapes=[pltpu.VMEM((N, N), F32)],
        compiler_params=pltpu.CompilerParams(
            dimension_semantics=("arbitrary",)),
    )(lf)


def _run_k3(lf, ef, category_ids, W3, b3, g3, bb3, W4, b4, g4, bb4,
            W5, b5, g5, bb5, W6, b6, g6, bb6, W7, b7, g7, bb7, cat_bias):
    BN_, D = lf.shape
    B = category_ids.shape[0]
    N = BN_ // B
    row = _row
    W3a, W3b = W3[:, :D], W3[:, D:]
    W7a, W7b = W7[:, :128], W7[:, 128:]
    el, v, mu7, rstd7, cbm = pl.pallas_call(
        _k3a_body,
        in_specs=[pl.BlockSpec(memory_space=pltpu.SMEM)] +
                 [pl.BlockSpec(memory_space=pltpu.VMEM)] * 23,
        out_shape=[
            jax.ShapeDtypeStruct((BN_, 128), F32),
            jax.ShapeDtypeStruct((B, 512), F32),
            jax.ShapeDtypeStruct((1, 512), F32),
            jax.ShapeDtypeStruct((1, 512), F32),
            jax.ShapeDtypeStruct((B, 512), F32),
        ],
    )(category_ids, lf, ef,
      W3a, W3b, row(b3), row(g3), row(bb3),
      W4, row(b4), row(g4), row(bb4),
      W5, row(b5), row(g5), row(bb5),
      W6, row(b6), row(g6), row(bb6),
      W7a, W7b, row(b7), cat_bias)

    out = pl.pallas_call(
        _k3b_body,
        grid=(B,),
        in_specs=[
            pl.BlockSpec((N, 128), lambda b: (b, 0)),
            pl.BlockSpec((1, 1, 512), lambda b: (b, 0, 0)),
            pl.BlockSpec((1, 512), lambda b: (0, 0)),
            pl.BlockSpec((1, 512), lambda b: (0, 0)),
            pl.BlockSpec((1, 512), lambda b: (0, 0)),
            pl.BlockSpec((1, 512), lambda b: (0, 0)),
            pl.BlockSpec((512, 128), lambda b: (0, 0)),
            pl.BlockSpec((1, 1, 512), lambda b: (b, 0, 0)),
        ],
        out_specs=pl.BlockSpec((N, 512), lambda b: (b, 0)),
        out_shape=jax.ShapeDtypeStruct((BN_, 512), F32),
        compiler_params=pltpu.CompilerParams(
            dimension_semantics=("arbitrary",)),
    )(el, v.reshape(B, 1, 512), mu7, rstd7, row(g7), row(bb7), W7a,
      cbm.reshape(B, 1, 512))
    return out


@jax.jit
def kernel(point_cloud, category_ids, W1, b1, g1, bb1, W2, b2, g2, bb2,
           W3, b3, g3, bb3, W4, b4, g4, bb4, W5, b5, g5, bb5,
           W6, b6, g6, bb6, W7, b7, g7, bb7, cat_bias):
    B, N, _ = point_cloud.shape
    x = point_cloud.reshape(B * N, 3)
    lf = _run_k1(x, W1, b1, g1, bb1, W2, b2, g2, bb2)
    ef = _run_k2(lf, B)
    out = _run_k3(lf, ef, category_ids, W3, b3, g3, bb3, W4, b4, g4, bb4,
                  W5, b5, g5, bb5, W6, b6, g6, bb6, W7, b7, g7, bb7,
                  cat_bias)
    return out.reshape(B, N, 512)
